# baseline (speedup 1.0000x reference)
import functools

import jax
import jax.numpy as jnp
from jax import lax
from jax.experimental import pallas as pl
from jax.experimental.pallas import tpu as pltpu
from jax.experimental.pallas import tpu_sc as plsc

B = 16384
F = 26
V = 100000
D = 64
VH0 = 51200
VB = 48768
TAIL = V - VH0 - VB
VH1 = VB + TAIL

NW = 32
PLANES = F * D
PPW = PLANES // NW
QB = B // 4

_mesh = plsc.VectorSubcoreMesh(core_axis_name="c", subcore_axis_name="s")


@functools.partial(
    pl.kernel,
    mesh=_mesh,
    compiler_params=pltpu.CompilerParams(needs_layout_passes=False),
    out_type=jax.ShapeDtypeStruct((F, D, B), jnp.float32),
    scratch_types=[
        pltpu.VMEM((VH0,), jnp.float32),
        pltpu.VMEM((VB,), jnp.float32),
        pltpu.VMEM((TAIL,), jnp.float32),
        pltpu.VMEM((B,), jnp.int32),
        pltpu.VMEM((QB,), jnp.float32),
        pltpu.VMEM((QB,), jnp.float32),
        pltpu.SemaphoreType.DMA,
        pltpu.SemaphoreType.DMA,
        pltpu.SemaphoreType.DMA,
        pltpu.SemaphoreType.DMA,
        pltpu.SemaphoreType.DMA,
        pltpu.SemaphoreType.DMA,
    ],
)
def _sc_plane_gather(tt_hbm, st_hbm, ttail_hbm, out_hbm,
                     h0, h1, tb, idx, ob0, ob1,
                     sem0, sem1, tsem, isem, ssem0, ssem1):
    wid = lax.axis_index("s") * 2 + lax.axis_index("c")
    p0 = wid * PPW
    f0 = lax.shift_right_logical(p0, 6)
    d0 = lax.bitwise_and(p0, D - 1)

    pltpu.async_copy(st_hbm.at[f0], idx, isem)
    pltpu.async_copy(tt_hbm.at[f0, d0].at[pl.ds(0, VH0)], h0, sem0)
    pltpu.async_copy(tt_hbm.at[f0, d0].at[pl.ds(VH0, VB)], h1, sem1)
    pltpu.async_copy(ttail_hbm.at[f0, d0], tb, tsem)

    def _passA(q, ob):
        @plsc.parallel_loop(0, QB // 16, unroll=16)
        def _vec(g):
            iv = idx[pl.ds(q * QB + g * 16, 16)]
            ivc = jnp.minimum(iv, VH0 - 1)
            ob[pl.ds(g * 16, 16)] = plsc.load_gather(h0, [ivc])

    def _passB(q, ob):
        @plsc.parallel_loop(0, QB // 16, unroll=16)
        def _vec(g):
            iv = idx[pl.ds(q * QB + g * 16, 16)]
            ivc = jnp.minimum(jnp.maximum(iv - VH0, 0), VB - 1)
            gb = plsc.load_gather(h1, [ivc])
            ivt = jnp.minimum(jnp.maximum(iv - (VH0 + VB), 0), TAIL - 1)
            gt = plsc.load_gather(tb, [ivt])
            gb = jnp.where(iv >= VH0 + VB, gt, gb)
            prev = ob[pl.ds(g * 16, 16)]
            ob[pl.ds(g * 16, 16)] = jnp.where(iv < VH0, prev, gb)

    def _plane(i, carry):
        p = p0 + i
        f = lax.shift_right_logical(p, 6)
        d = lax.bitwise_and(p, D - 1)

        pltpu.make_async_copy(
            tt_hbm.at[f, d].at[pl.ds(0, VH0)], h0, sem0).wait()

        @pl.when(jnp.logical_or(i == 0, d == 0))
        def _():
            pltpu.make_async_copy(st_hbm.at[f], idx, isem).wait()

        dr = [pltpu.make_async_copy(
                  ob, out_hbm.at[f, d, pl.ds(q * QB, QB)],
                  ssem0 if q % 2 == 0 else ssem1)
              for q, ob in ((0, ob0), (1, ob1), (2, ob0), (3, ob1))]

        for q, ob in ((0, ob0), (1, ob1)):
            @pl.when(i > 0)
            def _():
                dr[q].wait()
            _passA(q, ob)

        pltpu.make_async_copy(tt_hbm.at[f, d].at[pl.ds(VH0, VB)],
                              h1, sem1).wait()
        pltpu.make_async_copy(ttail_hbm.at[f, d], tb, tsem).wait()

        _passB(0, ob0)
        pltpu.async_copy(ob0, out_hbm.at[f, d, pl.ds(0, QB)], ssem0)
        _passB(1, ob1)
        pltpu.async_copy(ob1, out_hbm.at[f, d, pl.ds(QB, QB)], ssem1)

        dr[2].wait()
        _passA(2, ob0)
        dr[3].wait()
        _passA(3, ob1)

        pn = p + 1
        fn = lax.shift_right_logical(pn, 6)
        dn = lax.bitwise_and(pn, D - 1)

        @pl.when(i + 1 < PPW)
        def _():
            pltpu.async_copy(tt_hbm.at[fn, dn].at[pl.ds(0, VH0)], h0, sem0)

        _passB(2, ob0)
        pltpu.async_copy(ob0, out_hbm.at[f, d, pl.ds(2 * QB, QB)], ssem0)
        _passB(3, ob1)
        pltpu.async_copy(ob1, out_hbm.at[f, d, pl.ds(3 * QB, QB)], ssem1)

        @pl.when(i + 1 < PPW)
        def _():
            pltpu.async_copy(tt_hbm.at[fn, dn].at[pl.ds(VH0, VB)], h1, sem1)
            pltpu.async_copy(ttail_hbm.at[fn, dn], tb, tsem)

            @pl.when(dn == 0)
            def _():
                pltpu.async_copy(st_hbm.at[fn], idx, isem)

        return carry

    lax.fori_loop(0, PPW, _plane, 0)
    pltpu.make_async_copy(ob0, out_hbm.at[0, 0, pl.ds(0, QB)], ssem0).wait()
    pltpu.make_async_copy(ob1, out_hbm.at[0, 0, pl.ds(0, QB)], ssem1).wait()


def kernel(tables, s):
    tt = tables.transpose(0, 2, 1)
    st = s.T
    ttail = tt[:, :, VH0 + VB:]
    o = _sc_plane_gather(tt, st, ttail)
    return o.transpose(2, 0, 1)
